# baseline (speedup 1.0000x reference)
import jax
import jax.numpy as jnp
import numpy as np
from jax import lax
from jax.experimental import pallas as pl
from jax.experimental.pallas import tpu as pltpu
from jax.experimental.pallas import tpu_sc as plsc

F32 = jnp.float32

_GUMBEL_BITS = np.array([
    2098992034, -1375260455, -1648100871, -1885421097, 1935504149, -1778692392,
    321304473, -965794640, -1292960115, -778962000, 1504549425, -546233744,
    -318336956, -600734152, 2051079642, -779614296, -1239722701, -1775121607,
    316699916, -955105191, 1737595975, 511630552, 112767485, -1549715218,
    1738307256, -481718140, 349930173, 1273049434, -1689297813, -265278939,
    -1047937670, 1969816450, 1321672318, -1070068449, -1957936640, 1563429166,
    -1394101267, -1760446757, 1240164476, 810095772, 1784422759, -1627950868,
    1828558832, -1987278387, -1521476234, 623660575, -1406843393, -485275332,
    -1426729085, 1034178993, -1846804103, 1809243482, 1776712698, -426714319,
    -1519127926, 1843963808, 1740519301, 1464458439, -1532296560, -504388322,
    642185510, 917011611, -622342733, 1540263734], np.int32).reshape(8, 8)

_F32_TINY = np.finfo(np.float32).tiny

_NCHUNK = 4
_CROWS = 1024 // _NCHUNK


def _sc_body(arch_hbm, oph_hbm, node_hbm, tmf_hbm,
             xh_out, vmf_out,
             arch_v, vm_v, tm_v):
    pltpu.sync_copy(arch_hbm, arch_v)
    pltpu.sync_copy(tmf_hbm, tm_v)

    ops_vec = arch_v[pl.ds(0, 16)]
    pltpu.sync_copy(node_hbm, xh_out.at[pl.ds(0, 2)])
    for s in range(4):
        pltpu.sync_copy(oph_hbm.at[ops_vec[2 * s]],
                        xh_out.at[2 + s, pl.ds(0, 512)])
        pltpu.sync_copy(oph_hbm.at[ops_vec[2 * s + 1]],
                        xh_out.at[2 + s, pl.ds(512, 512)])
    pltpu.sync_copy(node_hbm.at[0], xh_out.at[6])
    pltpu.sync_copy(node_hbm.at[0], xh_out.at[7])

    for e in range(8):
        v16 = tm_v[pl.ds(ops_vec[e] * 8, 16)]
        vm_v[pl.ds(e * 8, 16)] = v16
    pltpu.sync_copy(vm_v.at[pl.ds(0, 64)], vmf_out)


def _tc_body(arch_smem, arch_ref, xh_ref, vm_ref, eaw_hbm, eab_ref,
             g1w_hbm, g1b_ref, g2w_hbm, g2b_ref, fcw_ref, fcb_ref,
             gbits_ref,
             arch_out, logp_out, ent_out, probt_out,
             w1_buf, w2_buf, w3_buf, sems):
    def _mk(wi, hbm, buf, ci):
        return pltpu.make_async_copy(
            hbm.at[pl.ds(ci * _CROWS, _CROWS), :],
            buf.at[pl.ds(ci * _CROWS, _CROWS), :],
            sems.at[wi, ci])

    w1_copies = [_mk(0, eaw_hbm, w1_buf, ci) for ci in range(_NCHUNK)]
    w2_copies = [_mk(1, g1w_hbm, w2_buf, ci) for ci in range(_NCHUNK)]
    w3_copies = [_mk(2, g2w_hbm, w3_buf, ci) for ci in range(_NCHUNK)]
    for cp in w1_copies:
        cp.start()
    w2_copies[0].start()
    w2_copies[1].start()

    row8 = lax.broadcasted_iota(jnp.int32, (8, 8), 0)
    col8 = lax.broadcasted_iota(jnp.int32, (8, 8), 1)

    def mm(a, b):
        return jnp.dot(a, b, preferred_element_type=F32)

    x_hidden = xh_ref[:]
    v_mask = vm_ref[:]
    adj = jnp.zeros((8, 8), F32)
    for e in range(8):
        f_e = lax.rem(arch_smem[e, 1], 6)
        t_e = lax.rem(arch_smem[e, 2], 6)
        adj = jnp.maximum(adj, jnp.where((row8 == t_e) & (col8 == f_e), 1.0, 0.0).astype(F32))

    def chunked_mm(a, buf, wcopies, prefetch):
        acc = None
        for ci in range(_NCHUNK):
            wcopies[ci].wait()
            if ci < len(prefetch):
                prefetch[ci].start()
            part = mm(a[:, ci * _CROWS:(ci + 1) * _CROWS],
                      buf[pl.ds(ci * _CROWS, _CROWS), :])
            acc = part if acc is None else acc + part
        return acc

    x = chunked_mm(x_hidden, w1_buf, w1_copies,
                   [w2_copies[2], w2_copies[3], w3_copies[0], w3_copies[1]]
                   ) + eab_ref[:]
    h1 = jnp.maximum(mm(adj, chunked_mm(x, w2_buf, w2_copies,
                                        [w3_copies[2], w3_copies[3]])
                        ) + g1b_ref[:], 0.0)
    h2 = mm(adj, chunked_mm(h1, w3_buf, w3_copies, [])) + g2b_ref[:]
    p_full = mm(h2, fcw_ref[:]) + fcb_ref[:]

    d_mat = jnp.where(col8 == 2 + row8 // 2, 1.0, 0.0).astype(F32)
    p_dup = mm(d_mat, p_full)
    parity = lax.rem(row8, 2)
    logits = jnp.where(parity == 0, p_dup[:, 0:8], p_dup[:, 8:16])

    z = logits - jnp.max(logits, axis=-1, keepdims=True)
    ez = jnp.exp(z) * v_mask
    prob = ez / jnp.sum(ez, axis=-1, keepdims=True)
    log_prob = jnp.log(jnp.clip(prob, 1e-5, 1.0 - 1e-5))
    entropy = -jnp.sum(log_prob * prob)

    fbits = lax.shift_right_logical(gbits_ref[:], 9) | jnp.int32(0x3F800000)
    floats = lax.bitcast_convert_type(fbits, F32) - F32(1.0)
    tiny = F32(_F32_TINY)
    unif = jnp.maximum(tiny, floats * (F32(1.0) - tiny) + tiny)
    gum = -jnp.log(-jnp.log(unif))
    samp_logits = jnp.where(v_mask > 0, jnp.log(jnp.clip(prob, 1e-12, 1.0)), -1e9)
    y = samp_logits + gum
    ymax = jnp.max(y, axis=-1, keepdims=True)
    fop = jnp.min(jnp.where(y == ymax, col8, 8), axis=-1, keepdims=True)
    logp = jnp.sum(jnp.where(col8 == fop, log_prob, 0.0))

    arch_out[:] = jnp.concatenate([fop, arch_ref[:, 1:3]], axis=1)
    logp_out[...] = logp
    ent_out[...] = entropy
    probt_out[:] = prob.T


def kernel(arch, node_hidden, op_hidden, emb_attn_w, emb_attn_b,
           gc1_w, gc1_b, gc2_w, gc2_b, fc_w, fc_b, trans_mask):
    arch = arch.astype(jnp.int32)

    mesh = plsc.VectorSubcoreMesh(core_axis_name="c", subcore_axis_name="s",
                                  num_cores=1, num_subcores=1)
    xh, vmf = pl.kernel(
        _sc_body,
        out_type=[
            jax.ShapeDtypeStruct((8, 1024), F32),
            jax.ShapeDtypeStruct((64,), F32),
        ],
        mesh=mesh,
        scratch_types=[
            pltpu.VMEM((48,), jnp.int32),
            pltpu.VMEM((128,), F32),
            pltpu.VMEM((128,), F32),
        ],
    )(jnp.pad(arch.T, ((0, 0), (0, 8))).reshape(-1), op_hidden, node_hidden,
      jnp.pad(trans_mask.reshape(-1), (0, 64)))

    vmem = pl.BlockSpec(memory_space=pltpu.VMEM)
    smem = pl.BlockSpec(memory_space=pltpu.SMEM)
    anym = pl.BlockSpec(memory_space=pltpu.MemorySpace.HBM)
    new_arch, logp, ent, probt = pl.pallas_call(
        _tc_body,
        in_specs=[smem, vmem, vmem, vmem, anym, vmem, anym, vmem, anym,
                  vmem, vmem, vmem, vmem],
        out_specs=[vmem, smem, smem, vmem],
        out_shape=[
            jax.ShapeDtypeStruct((8, 3), jnp.int32),
            jax.ShapeDtypeStruct((), F32),
            jax.ShapeDtypeStruct((), F32),
            jax.ShapeDtypeStruct((8, 8), F32),
        ],
        scratch_shapes=[
            pltpu.VMEM((1024, 1024), F32),
            pltpu.VMEM((1024, 1024), F32),
            pltpu.VMEM((1024, 1024), F32),
            pltpu.SemaphoreType.DMA((3, _NCHUNK)),
        ],
    )(arch, arch, xh, vmf.reshape(8, 8), emb_attn_w,
      emb_attn_b.reshape(1, -1), gc1_w, gc1_b.reshape(1, -1),
      gc2_w, gc2_b.reshape(1, -1), fc_w, fc_b.reshape(1, -1),
      jnp.asarray(_GUMBEL_BITS))

    return (new_arch, logp, ent, probt)


# speedup vs baseline: 3.2994x; 3.2994x over previous
import jax
import jax.numpy as jnp
import numpy as np
from jax import lax
from jax.experimental import pallas as pl
from jax.experimental.pallas import tpu as pltpu

F32 = jnp.float32

_GUMBEL_BITS = np.array([
    2098992034, -1375260455, -1648100871, -1885421097, 1935504149, -1778692392,
    321304473, -965794640, -1292960115, -778962000, 1504549425, -546233744,
    -318336956, -600734152, 2051079642, -779614296, -1239722701, -1775121607,
    316699916, -955105191, 1737595975, 511630552, 112767485, -1549715218,
    1738307256, -481718140, 349930173, 1273049434, -1689297813, -265278939,
    -1047937670, 1969816450, 1321672318, -1070068449, -1957936640, 1563429166,
    -1394101267, -1760446757, 1240164476, 810095772, 1784422759, -1627950868,
    1828558832, -1987278387, -1521476234, 623660575, -1406843393, -485275332,
    -1426729085, 1034178993, -1846804103, 1809243482, 1776712698, -426714319,
    -1519127926, 1843963808, 1740519301, 1464458439, -1532296560, -504388322,
    642185510, 917011611, -622342733, 1540263734], np.int32).reshape(8, 8)

_F32_TINY = np.finfo(np.float32).tiny


_NCHUNK = 4
_CROWS = 1024 // _NCHUNK


def _fused_body(arch_smem, arch_ref, node_ref, oph_ref, eaw_hbm, eab_ref,
                g1w_hbm, g1b_ref, g2w_hbm, g2b_ref, fcw_ref, fcb_ref,
                tm_ref, gbits_ref,
                arch_out, logp_out, ent_out, probt_out,
                w1_buf, w2_buf, w3_buf, sems):
    def _mk(wi, hbm, buf, ci):
        return pltpu.make_async_copy(
            hbm.at[pl.ds(ci * _CROWS, _CROWS), :],
            buf.at[pl.ds(ci * _CROWS, _CROWS), :],
            sems.at[wi, ci])

    w1_copies = [_mk(0, eaw_hbm, w1_buf, ci) for ci in range(_NCHUNK)]
    w2_copies = [_mk(1, g1w_hbm, w2_buf, ci) for ci in range(_NCHUNK)]
    w3_copies = [_mk(2, g2w_hbm, w3_buf, ci) for ci in range(_NCHUNK)]
    for cp in w1_copies:
        cp.start()
    w2_copies[0].start()
    w2_copies[1].start()

    row8 = lax.broadcasted_iota(jnp.int32, (8, 8), 0)
    col8 = lax.broadcasted_iota(jnp.int32, (8, 8), 1)

    sel = jnp.zeros((8, 8), F32)
    sel_even = jnp.zeros((8, 8), F32)
    sel_odd = jnp.zeros((8, 8), F32)
    adj = jnp.zeros((8, 8), F32)
    for e in range(8):
        op_e = arch_smem[e, 0]
        f_e = lax.rem(arch_smem[e, 1], 6)
        t_e = lax.rem(arch_smem[e, 2], 6)
        sel = sel + jnp.where((row8 == e) & (col8 == op_e), 1.0, 0.0).astype(F32)
        hit = jnp.where((row8 == 2 + e // 2) & (col8 == op_e), 1.0, 0.0).astype(F32)
        if e % 2 == 0:
            sel_even = sel_even + hit
        else:
            sel_odd = sel_odd + hit
        adj = jnp.maximum(adj, jnp.where((row8 == t_e) & (col8 == f_e), 1.0, 0.0).astype(F32))

    def mm(a, b):
        return jnp.dot(a, b, preferred_element_type=F32)

    x_left = mm(sel_even, oph_ref[:])
    x_right = mm(sel_odd, oph_ref[:])
    node_pad = jnp.concatenate([node_ref[:], jnp.zeros((6, 1024), F32)], axis=0)
    x_hidden = node_pad + jnp.concatenate([x_left, x_right], axis=1)

    def chunked_mm(a, buf, wcopies, prefetch):
        acc = None
        for ci in range(_NCHUNK):
            wcopies[ci].wait()
            if ci < len(prefetch):
                prefetch[ci].start()
            part = mm(a[:, ci * _CROWS:(ci + 1) * _CROWS],
                      buf[pl.ds(ci * _CROWS, _CROWS), :])
            acc = part if acc is None else acc + part
        return acc

    x = chunked_mm(x_hidden, w1_buf, w1_copies,
                   [w2_copies[2], w2_copies[3], w3_copies[0], w3_copies[1]]
                   ) + eab_ref[:]
    h1 = jnp.maximum(mm(adj, chunked_mm(x, w2_buf, w2_copies,
                                        [w3_copies[2], w3_copies[3]])
                        ) + g1b_ref[:], 0.0)
    h2 = mm(adj, chunked_mm(h1, w3_buf, w3_copies, [])) + g2b_ref[:]
    p_full = mm(h2, fcw_ref[:]) + fcb_ref[:]

    d_mat = jnp.where(col8 == 2 + row8 // 2, 1.0, 0.0).astype(F32)
    p_dup = mm(d_mat, p_full)
    parity = lax.rem(row8, 2)
    logits = jnp.where(parity == 0, p_dup[:, 0:8], p_dup[:, 8:16])

    v_mask = mm(sel, tm_ref[:])
    z = logits - jnp.max(logits, axis=-1, keepdims=True)
    ez = jnp.exp(z) * v_mask
    prob = ez / jnp.sum(ez, axis=-1, keepdims=True)
    log_prob = jnp.log(jnp.clip(prob, 1e-5, 1.0 - 1e-5))
    entropy = -jnp.sum(log_prob * prob)

    fbits = lax.shift_right_logical(gbits_ref[:], 9) | jnp.int32(0x3F800000)
    floats = lax.bitcast_convert_type(fbits, F32) - F32(1.0)
    tiny = F32(_F32_TINY)
    unif = jnp.maximum(tiny, floats * (F32(1.0) - tiny) + tiny)
    gum = -jnp.log(-jnp.log(unif))
    samp_logits = jnp.where(v_mask > 0, jnp.log(jnp.clip(prob, 1e-12, 1.0)), -1e9)
    y = samp_logits + gum
    ymax = jnp.max(y, axis=-1, keepdims=True)
    fop = jnp.min(jnp.where(y == ymax, col8, 8), axis=-1, keepdims=True)
    logp = jnp.sum(jnp.where(col8 == fop, log_prob, 0.0))

    arch_out[:] = jnp.concatenate([fop, arch_ref[:, 1:3]], axis=1)
    logp_out[...] = logp
    ent_out[...] = entropy
    probt_out[:] = prob.T


def kernel(arch, node_hidden, op_hidden, emb_attn_w, emb_attn_b,
           gc1_w, gc1_b, gc2_w, gc2_b, fc_w, fc_b, trans_mask):
    arch = arch.astype(jnp.int32)
    vmem = pl.BlockSpec(memory_space=pltpu.VMEM)
    smem = pl.BlockSpec(memory_space=pltpu.SMEM)
    anym = pl.BlockSpec(memory_space=pltpu.MemorySpace.HBM)
    new_arch, logp, ent, probt = pl.pallas_call(
        _fused_body,
        in_specs=[smem, vmem, vmem, vmem, anym, vmem, anym, vmem, anym,
                  vmem, vmem, vmem, vmem, vmem],
        out_specs=[vmem, smem, smem, vmem],
        out_shape=[
            jax.ShapeDtypeStruct((8, 3), jnp.int32),
            jax.ShapeDtypeStruct((), F32),
            jax.ShapeDtypeStruct((), F32),
            jax.ShapeDtypeStruct((8, 8), F32),
        ],
        scratch_shapes=[
            pltpu.VMEM((1024, 1024), F32),
            pltpu.VMEM((1024, 1024), F32),
            pltpu.VMEM((1024, 1024), F32),
            pltpu.SemaphoreType.DMA((3, _NCHUNK)),
        ],
    )(arch, arch, node_hidden, op_hidden, emb_attn_w,
      emb_attn_b.reshape(1, -1), gc1_w, gc1_b.reshape(1, -1),
      gc2_w, gc2_b.reshape(1, -1), fc_w, fc_b.reshape(1, -1),
      trans_mask, jnp.asarray(_GUMBEL_BITS))

    return (new_arch, logp, ent, probt)
